# baseline (speedup 1.0000x reference)
import functools

import jax
import jax.numpy as jnp
from jax import lax
from jax.experimental import pallas as pl
from jax.experimental.pallas import tpu as pltpu
from jax.experimental.pallas import tpu_sc as plsc

_LANES = 16
_BB = 4
_NBUF = 8
_DEPTH = 4
_SUB = 8
_LNS = 128


@functools.lru_cache(maxsize=None)
def _build_lookup(B, F, V, D):
    info = plsc.get_sparse_core_info()
    NC, NS = info.num_cores, info.num_subcores
    NW = NC * NS
    b_per_w = B // NW
    chunk = b_per_w * F
    rows = _BB * F
    n_batches = b_per_w // _BB
    FP = (F + _SUB - 1) // _SUB
    CW = b_per_w // _LNS
    assert B % (NW * _LNS) == 0 and chunk % _LANES == 0 and rows <= 128
    assert rows % 8 == 0 and n_batches % _NBUF == 0 and n_batches >= 2 * _NBUF
    mesh = plsc.VectorSubcoreMesh(core_axis_name="c", subcore_axis_name="s")

    @functools.partial(
        pl.kernel,
        mesh=mesh,
        out_type=jax.ShapeDtypeStruct((B, F, D), jnp.float32),
        scratch_types=[
            pltpu.VMEM((FP, CW, _SUB, _LNS), jnp.int32),
            pltpu.VMEM((chunk,), jnp.int32),
            pltpu.VMEM((_NBUF, rows, D), jnp.float32),
            pltpu.SemaphoreType.DMA((_NBUF,)),
            pltpu.SemaphoreType.DMA((_NBUF,)),
        ],
        compiler_params=pltpu.CompilerParams(
            use_tc_tiling_on_sc=False, needs_layout_passes=False
        ),
    )
    def lookup(x_hbm, tab_hbm, out_hbm, x_v, idx_v, rows_v, gsem, osem):
        wid = lax.axis_index("s") * NC + lax.axis_index("c")
        bsamp = wid * b_per_w
        pltpu.sync_copy(x_hbm.at[:, pl.ds(wid * CW, CW)], x_v)

        _MAGIC, _SHIFT = (1 << 19) // F + 1, 19

        def add_offsets(i, carry):
            p = i * _LANES + lax.iota(jnp.int32, _LANES)
            bloc = lax.shift_right_logical(p * _MAGIC, _SHIFT)
            f = p - bloc * F
            v = plsc.load_gather(
                x_v,
                [
                    lax.shift_right_logical(f, 3),
                    lax.shift_right_logical(bloc, 7),
                    lax.bitwise_and(f, 7),
                    lax.bitwise_and(bloc, 127),
                ],
            )
            idx_v[pl.ds(i * _LANES, _LANES)] = v + f * V
            return carry

        lax.fori_loop(0, chunk // _LANES, add_offsets, 0)

        def gather(j, b):
            pltpu.async_copy(
                tab_hbm.at[idx_v.at[pl.ds(j * rows, rows)]],
                rows_v.at[b],
                gsem.at[b],
            )

        def wait_gather(b):
            pltpu.make_async_copy(
                tab_hbm.at[pl.ds(0, rows)], rows_v.at[b], gsem.at[b]
            ).wait()

        def copy_out(j, b):
            for k in range(_BB):
                pltpu.async_copy(
                    rows_v.at[b, pl.ds(k * F, F)],
                    out_hbm.at[bsamp + j * _BB + k],
                    osem.at[b],
                )

        def wait_copy_out(b):
            pltpu.make_async_copy(
                tab_hbm.at[pl.ds(0, rows)], rows_v.at[b], osem.at[b]
            ).wait()

        for b in range(_DEPTH):
            gather(b, b)

        def outer(g, carry):
            for b in range(_NBUF):
                j = g * _NBUF + b
                wait_gather(b)
                copy_out(j, b)
                j2 = j + _DEPTH
                b2 = (b + _DEPTH) % _NBUF

                @pl.when(j2 < n_batches)
                def _():
                    @pl.when(j2 >= _NBUF)
                    def _():
                        wait_copy_out(b2)

                    gather(j2, b2)

            return carry

        lax.fori_loop(0, n_batches // _NBUF, outer, 0)

        for b in range(_NBUF):
            wait_copy_out(b)

    return lookup


def kernel(x, tables):
    B, F = x.shape
    Ft, V, D = tables.shape
    FP = (F + _SUB - 1) // _SUB
    xp = jnp.pad(x.T, ((0, FP * _SUB - F), (0, 0)))
    xt = xp.reshape(FP, _SUB, B // _LNS, _LNS).transpose(0, 2, 1, 3)
    tab_flat = tables.reshape(Ft * V, D)
    return _build_lookup(B, F, V, D)(xt, tab_flat)


# speedup vs baseline: 1.0405x; 1.0405x over previous
import functools

import jax
import jax.numpy as jnp
from jax import lax
from jax.experimental import pallas as pl
from jax.experimental.pallas import tpu as pltpu
from jax.experimental.pallas import tpu_sc as plsc

_NBUF = 8
_DEPTH = 4
_SUB = 8
_LNS = 128


@functools.lru_cache(maxsize=None)
def _build_lookup(B, F, V, D):
    info = plsc.get_sparse_core_info()
    NC, NS = info.num_cores, info.num_subcores
    NW = NC * NS
    b_per_w = B // NW
    CW = b_per_w // _LNS
    FP = (F + _SUB - 1) // _SUB
    n_batches = F * CW
    assert B % (NW * _LNS) == 0 and CW & (CW - 1) == 0
    assert n_batches % _NBUF == 0 and n_batches >= 2 * _NBUF
    cw_sh = CW.bit_length() - 1
    mesh = plsc.VectorSubcoreMesh(core_axis_name="c", subcore_axis_name="s")

    @functools.partial(
        pl.kernel,
        mesh=mesh,
        out_type=jax.ShapeDtypeStruct((F, B, D), jnp.float32),
        scratch_types=[
            pltpu.VMEM((FP, CW, _SUB, _LNS), jnp.int32),
            pltpu.VMEM((_NBUF, _LNS, D), jnp.float32),
            pltpu.SemaphoreType.DMA((_NBUF,)),
            pltpu.SemaphoreType.DMA((_NBUF,)),
        ],
        compiler_params=pltpu.CompilerParams(
            use_tc_tiling_on_sc=False, needs_layout_passes=False
        ),
    )
    def lookup(x_hbm, tab_hbm, out_hbm, x_v, rows_v, gsem, osem):
        wid = lax.axis_index("s") * NC + lax.axis_index("c")
        b0 = wid * b_per_w
        pltpu.sync_copy(x_hbm.at[:, pl.ds(wid * CW, CW)], x_v)

        def coords(j):
            f = lax.shift_right_logical(j, cw_sh)
            c = lax.bitwise_and(j, CW - 1)
            return f, c

        def gather(j, b):
            f, c = coords(j)
            idx = x_v.at[
                lax.shift_right_logical(f, 3), c, lax.bitwise_and(f, 7)
            ]
            pltpu.async_copy(
                tab_hbm.at[f].at[idx],
                rows_v.at[b],
                gsem.at[b],
            )

        def wait_gather(b):
            pltpu.make_async_copy(
                tab_hbm.at[0].at[pl.ds(0, _LNS)], rows_v.at[b], gsem.at[b]
            ).wait()

        def copy_out(j, b):
            f, c = coords(j)
            pltpu.async_copy(
                rows_v.at[pl.ds(b, 1)],
                out_hbm.at[pl.ds(f, 1), pl.ds(b0 + c * _LNS, _LNS)],
                osem.at[b],
            )

        def wait_copy_out(b):
            pltpu.make_async_copy(
                tab_hbm.at[0].at[pl.ds(0, _LNS)], rows_v.at[b], osem.at[b]
            ).wait()

        for b in range(_DEPTH):
            gather(b, b)

        def outer(g, carry):
            for b in range(_NBUF):
                j = g * _NBUF + b
                wait_gather(b)
                copy_out(j, b)
                j2 = j + _DEPTH
                b2 = (b + _DEPTH) % _NBUF

                @pl.when(j2 < n_batches)
                def _():
                    @pl.when(j2 >= _NBUF)
                    def _():
                        wait_copy_out(b2)

                    gather(j2, b2)

            return carry

        lax.fori_loop(0, n_batches // _NBUF, outer, 0)

        for b in range(_NBUF):
            wait_copy_out(b)

    return lookup


def kernel(x, tables):
    B, F = x.shape
    Ft, V, D = tables.shape
    FP = (F + _SUB - 1) // _SUB
    xp = jnp.pad(x.T, ((0, FP * _SUB - F), (0, 0)))
    xt = xp.reshape(FP, _SUB, B // _LNS, _LNS).transpose(0, 2, 1, 3)
    out_fbd = _build_lookup(B, F, V, D)(xt, tables)
    return out_fbd.transpose(1, 0, 2)
